# Initial kernel scaffold; baseline (speedup 1.0000x reference)
import jax
import jax.numpy as jnp
from jax.experimental import pallas as pl


def kernel(x, emb_table, pos_table):
    raise NotImplementedError("write your pallas kernel here")


# speedup vs baseline: 2.6021x; 2.6021x over previous
import functools

import jax
import jax.numpy as jnp
from jax import lax
from jax.experimental import pallas as pl
from jax.experimental.pallas import tpu as pltpu
from jax.experimental.pallas import tpu_sc as plsc


def _make_emb_kernel(B, L, H, V):
    info = plsc.get_sparse_core_info()
    NC, NS, LN = info.num_cores, info.num_subcores, info.num_lanes
    NW = NC * NS
    assert B % NW == 0 and H % LN == 0
    rows_per_w = B // NW
    chunks = []
    off = 0
    while off < L:
        sz = min(128, L - off)
        chunks.append((off, sz))
        off += sz

    mesh = plsc.VectorSubcoreMesh(core_axis_name="c", subcore_axis_name="s")

    @functools.partial(
        pl.kernel,
        out_type=jax.ShapeDtypeStruct((B, L, H), jnp.float32),
        mesh=mesh,
        scratch_types=[
            pltpu.VMEM((L,), jnp.int32),
            pltpu.VMEM((L, H), jnp.float32),
            pltpu.VMEM((L, H), jnp.float32),
            pltpu.SemaphoreType.DMA,
        ],
        compiler_params=pltpu.CompilerParams(use_tc_tiling_on_sc=False),
    )
    def emb_kernel(x_hbm, emb_hbm, pos_hbm, out_hbm, idx_v, rows_v, pos_v, sem):
        wid = lax.axis_index("s") * NC + lax.axis_index("c")
        pltpu.sync_copy(pos_hbm, pos_v)

        def row_body(i, carry):
            b = wid * rows_per_w + i
            pltpu.sync_copy(x_hbm.at[b], idx_v)
            cps = [
                pltpu.async_copy(
                    emb_hbm.at[idx_v.at[pl.ds(off, sz)]],
                    rows_v.at[pl.ds(off, sz)],
                    sem,
                )
                for off, sz in chunks
            ]
            for cp in cps:
                cp.wait()

            def add_body(r, c2):
                for c4 in range(H // LN):
                    sl = pl.ds(c4 * LN, LN)
                    plsc.addupdate(rows_v.at[r, sl], pos_v[r, sl])
                return c2

            lax.fori_loop(0, L, add_body, 0)
            pltpu.sync_copy(rows_v, out_hbm.at[b])
            return carry

        lax.fori_loop(0, rows_per_w, row_body, 0)

    return emb_kernel


def kernel(x, emb_table, pos_table):
    B, L = x.shape
    V, H = emb_table.shape
    emb = _make_emb_kernel(B, L, H, V)
    return emb(x.astype(jnp.int32), emb_table, pos_table)


# speedup vs baseline: 2.8227x; 1.0848x over previous
import functools

import jax
import jax.numpy as jnp
from jax import lax
from jax.experimental import pallas as pl
from jax.experimental.pallas import tpu as pltpu
from jax.experimental.pallas import tpu_sc as plsc

_LANE = 128


def _make_emb_kernel(B, L, H, V):
    info = plsc.get_sparse_core_info()
    NC, NS, LN = info.num_cores, info.num_subcores, info.num_lanes
    NW = NC * NS
    assert B % NW == 0 and H % LN == 0
    rows_per_w = B // NW
    chunks = []
    off = 0
    while off < L:
        sz = min(128, L - off)
        chunks.append((off, sz))
        off += sz

    mesh = plsc.VectorSubcoreMesh(core_axis_name="c", subcore_axis_name="s")

    @functools.partial(
        pl.kernel,
        out_type=jax.ShapeDtypeStruct((B, L, H), jnp.float32),
        mesh=mesh,
        scratch_types=[
            pltpu.VMEM((L,), jnp.int32),
            pltpu.VMEM((L, _LANE), jnp.float32),
            pltpu.VMEM((L, _LANE), jnp.float32),
            pltpu.VMEM((L, H), jnp.float32),
            pltpu.SemaphoreType.DMA,
        ],
    )
    def emb_kernel(x_hbm, emb_hbm, pos_hbm, out_hbm, idx_v, pos_v, rows_v,
                   out_v, sem):
        wid = lax.axis_index("s") * NC + lax.axis_index("c")
        pltpu.sync_copy(pos_hbm, pos_v)

        def row_body(i, carry):
            b = wid * rows_per_w + i
            pltpu.sync_copy(x_hbm.at[pl.ds(b * L, L)], idx_v)
            cps = [
                pltpu.async_copy(
                    emb_hbm.at[idx_v.at[pl.ds(off, sz)]],
                    rows_v.at[pl.ds(off, sz)],
                    sem,
                )
                for off, sz in chunks
            ]
            for cp in cps:
                cp.wait()

            def add_body(r, c2):
                for c4 in range(H // LN):
                    sl = pl.ds(c4 * LN, LN)
                    out_v[r, sl] = rows_v[r, sl] + pos_v[r, sl]
                return c2

            lax.fori_loop(0, L, add_body, 0)
            pltpu.sync_copy(out_v, out_hbm.at[b])
            return carry

        lax.fori_loop(0, rows_per_w, row_body, 0)

    return emb_kernel


def kernel(x, emb_table, pos_table):
    B, L = x.shape
    V, H = emb_table.shape
    emb_pad = jnp.pad(emb_table, ((0, 0), (0, _LANE - H)))
    pos_pad = jnp.pad(pos_table, ((0, 0), (0, _LANE - H)))
    x_flat = jnp.reshape(x.astype(jnp.int32), (-1,))
    emb = _make_emb_kernel(B, L, H, V)
    return emb(x_flat, emb_pad, pos_pad)


# speedup vs baseline: 2.9069x; 1.0298x over previous
import functools

import jax
import jax.numpy as jnp
from jax import lax
from jax.experimental import pallas as pl
from jax.experimental.pallas import tpu as pltpu
from jax.experimental.pallas import tpu_sc as plsc

_LANE = 128


def _make_emb_kernel(B, L, H, V):
    info = plsc.get_sparse_core_info()
    NC, NS, LN = info.num_cores, info.num_subcores, info.num_lanes
    NW = NC * NS
    T = B * L
    assert T % (NW * _LANE) == 0 and H % LN == 0
    chunks_per_w = T // (NW * _LANE)
    rows_per_w = chunks_per_w * _LANE // _LANE

    mesh = plsc.VectorSubcoreMesh(core_axis_name="c", subcore_axis_name="s")

    @functools.partial(
        pl.kernel,
        out_type=jax.ShapeDtypeStruct((T // _LANE, _LANE, H), jnp.float32),
        mesh=mesh,
        scratch_types=[
            pltpu.VMEM((chunks_per_w * _LANE,), jnp.int32),
            pltpu.VMEM((L, _LANE), jnp.float32),
            pltpu.VMEM((_LANE, _LANE), jnp.float32),
            pltpu.VMEM((_LANE, _LANE), jnp.float32),
            pltpu.VMEM((_LANE, H), jnp.float32),
            pltpu.SemaphoreType.DMA,
            pltpu.SemaphoreType.DMA,
        ],
    )
    def emb_kernel(x_hbm, emb_hbm, pos_hbm, out_hbm, idx_v, pos_v, buf_a,
                   buf_b, out_v, sem_a, sem_b):
        wid = lax.axis_index("s") * NC + lax.axis_index("c")
        tok0 = wid * chunks_per_w * _LANE
        pltpu.sync_copy(pos_hbm, pos_v)
        pltpu.sync_copy(x_hbm.at[pl.ds(tok0, chunks_per_w * _LANE)], idx_v)

        def start_gather(j, buf, sem):
            off = pl.multiple_of(j * _LANE, _LANE)
            pltpu.async_copy(emb_hbm.at[idx_v.at[pl.ds(off, _LANE)]], buf, sem)

        def wait_gather(buf, sem):
            pltpu.make_async_copy(
                emb_hbm.at[idx_v.at[pl.ds(0, _LANE)]], buf, sem).wait()

        def add_and_store(j, buf):
            base = pl.multiple_of(tok0 + j * _LANE, _LANE)

            def add_body(r, carry):
                p = lax.rem(base + r, L)
                for c4 in range(H // LN):
                    sl = pl.ds(c4 * LN, LN)
                    out_v[r, sl] = buf[r, sl] + pos_v[p, sl]
                return carry

            lax.fori_loop(0, _LANE, add_body, 0)
            pltpu.sync_copy(out_v, out_hbm.at[wid * chunks_per_w + j])

        start_gather(0, buf_a, sem_a)

        def pair_body(j2, carry):
            j0 = 2 * j2

            @pl.when(j0 + 1 < chunks_per_w)
            def _():
                start_gather(j0 + 1, buf_b, sem_b)

            wait_gather(buf_a, sem_a)
            add_and_store(j0, buf_a)

            @pl.when(j0 + 2 < chunks_per_w)
            def _():
                start_gather(j0 + 2, buf_a, sem_a)

            @pl.when(j0 + 1 < chunks_per_w)
            def _():
                wait_gather(buf_b, sem_b)
                add_and_store(j0 + 1, buf_b)

            return carry

        lax.fori_loop(0, (chunks_per_w + 1) // 2, pair_body, 0)

    return emb_kernel


def kernel(x, emb_table, pos_table):
    B, L = x.shape
    V, H = emb_table.shape
    emb_pad = jnp.pad(emb_table, ((0, 0), (0, _LANE - H)))
    pos_pad = jnp.pad(pos_table, ((0, 0), (0, _LANE - H)))
    x_flat = jnp.reshape(x.astype(jnp.int32), (-1,))
    emb = _make_emb_kernel(B, L, H, V)
    out = emb(x_flat, emb_pad, pos_pad)
    return jnp.reshape(out, (B, L, H))
